# baseline (speedup 1.0000x reference)
import functools

import jax
import jax.numpy as jnp
from jax import lax
from jax.experimental import pallas as pl
from jax.experimental.pallas import tpu as pltpu
from jax.experimental.pallas import tpu_sc as plsc

B, S, D, V = 4096, 200, 128, 100000
NC, NS, L = 2, 16, 16
NW = NC * NS
ROWS = B * S
RPW = ROWS // NW
CH = 256
CPW = RPW // CH
NB = 3


def _emb_body(ids_hbm, pos_hbm, table_hbm, out_hbm,
              pos_v, i0, i1, i2, r0, r1, r2,
              is0, is1, is2, gs0, gs1, gs2, os0, os1, os2):
    idxs = (i0, i1, i2)
    rb = (r0, r1, r2)
    isems = (is0, is1, is2)
    gsems = (gs0, gs1, gs2)
    osems = (os0, os1, os2)

    wid = lax.axis_index("s") * NC + lax.axis_index("c")
    f0 = wid * RPW
    pltpu.sync_copy(pos_hbm, pos_v)

    def idx_start(r, c):
        pltpu.async_copy(ids_hbm.at[pl.ds(f0 + c * CH, CH)], idxs[r], isems[r])

    def idx_wait(r):
        pltpu.make_async_copy(ids_hbm.at[pl.ds(f0, CH)], idxs[r],
                              isems[r]).wait()

    def gather_start(r):
        for h in range(2):
            pltpu.async_copy(table_hbm.at[idxs[r].at[pl.ds(h * 128, 128)]],
                             rb[r].at[pl.ds(h * 128, 128)], gsems[r])

    def gather_wait(r):
        for h in range(2):
            pltpu.make_async_copy(table_hbm.at[idxs[r].at[pl.ds(h * 128, 128)]],
                                  rb[r].at[pl.ds(h * 128, 128)],
                                  gsems[r]).wait()

    def out_start(r, c):
        pltpu.async_copy(rb[r], out_hbm.at[pl.ds(f0 + c * CH, CH)], osems[r])

    def out_wait(r):
        pltpu.make_async_copy(rb[r], out_hbm.at[pl.ds(f0, CH)],
                              osems[r]).wait()

    idx_start(0, 0)
    idx_start(1, 1)
    idx_wait(0)
    gather_start(0)

    def iter_body(c, r):
        rg = (r + 1) % NB
        ri = (r + 2) % NB

        def prefetch_idx():
            idx_start(ri, c + 2)
        pl.when(c + 2 < CPW)(prefetch_idx)

        def start_next_gather():
            def drain_out():
                out_wait(rg)
            pl.when(c >= 2)(drain_out)
            idx_wait(rg)
            gather_start(rg)
        pl.when(c + 1 < CPW)(start_next_gather)

        gather_wait(r)

        s0 = lax.rem(c * CH, S)

        @plsc.parallel_loop(0, CH, unroll=4)
        def addrow(row):
            srow = lax.rem(s0 + row, S)
            for p in range(D // L):
                sl = pl.ds(p * L, L)
                rb[r][row, sl] = rb[r][row, sl] + pos_v[srow, sl]

        out_start(r, c)

    def outer(g, carry):
        for r in range(NB):
            iter_body(NB * g + r, r)
        return carry

    lax.fori_loop(0, CPW // NB, outer, 0)
    iter_body(CPW - 1, (CPW - 1) % NB)
    for r in range(NB):
        out_wait(r)


@jax.jit
def kernel(input_ids, table, pos_embed):
    mesh = plsc.VectorSubcoreMesh(core_axis_name="c", subcore_axis_name="s")
    out = pl.kernel(
        _emb_body,
        mesh=mesh,
        out_type=jax.ShapeDtypeStruct((ROWS, D), jnp.float32),
        scratch_types=(
            [pltpu.VMEM((S, D), jnp.float32)]
            + [pltpu.VMEM((CH,), jnp.int32)] * NB
            + [pltpu.VMEM((CH, D), jnp.float32)] * NB
            + [pltpu.SemaphoreType.DMA] * (3 * NB)
        ),
    )(input_ids.reshape(ROWS), pos_embed.reshape(S, D), table)
    return out.reshape(B, S, D)


# speedup vs baseline: 1.0247x; 1.0247x over previous
import functools

import jax
import jax.numpy as jnp
from jax import lax
from jax.experimental import pallas as pl
from jax.experimental.pallas import tpu as pltpu
from jax.experimental.pallas import tpu_sc as plsc

B, S, D, V = 4096, 200, 128, 100000
NC, NS, L = 2, 16, 16
NW = NC * NS
BPW = B // NW
SPLIT = 104


def _emb_body(ids_hbm, pos_hbm, table_hbm, out_hbm,
              pos_v, idx0, idx1, g0, g1, o0, o1,
              isem0, isem1, gsem0, gsem1, osem0, osem1):
    idxs = (idx0, idx1)
    gb = (g0, g1)
    ob = (o0, o1)
    isems = (isem0, isem1)
    gsems = (gsem0, gsem1)
    osems = (osem0, osem1)

    wid = lax.axis_index("s") * NC + lax.axis_index("c")
    b0 = wid * BPW
    pltpu.sync_copy(pos_hbm.at[0], pos_v)

    def idx_start(k, b):
        pltpu.async_copy(ids_hbm.at[b], idxs[k], isems[k])

    def idx_wait(k):
        pltpu.make_async_copy(ids_hbm.at[b0], idxs[k], isems[k]).wait()

    def gather_start(k):
        pltpu.async_copy(table_hbm.at[idxs[k].at[pl.ds(0, SPLIT)]],
                         gb[k].at[pl.ds(0, SPLIT)], gsems[k])
        pltpu.async_copy(table_hbm.at[idxs[k].at[pl.ds(SPLIT, S - SPLIT)]],
                         gb[k].at[pl.ds(SPLIT, S - SPLIT)], gsems[k])

    def gather_wait(k):
        pltpu.make_async_copy(table_hbm.at[idxs[k].at[pl.ds(0, SPLIT)]],
                              gb[k].at[pl.ds(0, SPLIT)], gsems[k]).wait()
        pltpu.make_async_copy(table_hbm.at[idxs[k].at[pl.ds(SPLIT, S - SPLIT)]],
                              gb[k].at[pl.ds(SPLIT, S - SPLIT)], gsems[k]).wait()

    def out_start(k, b):
        pltpu.async_copy(ob[k], out_hbm.at[b], osems[k])

    def out_wait(k):
        pltpu.make_async_copy(ob[k], out_hbm.at[b0], osems[k]).wait()

    idx_start(0, b0)
    idx_wait(0)
    gather_start(0)
    idx_start(1, b0 + 1)

    def iter_body(i, k):
        k2 = 1 - k

        def start_next_gather():
            idx_wait(k2)
            gather_start(k2)
        pl.when(i + 1 < BPW)(start_next_gather)

        gather_wait(k)

        def prefetch_idx():
            idx_start(k, b0 + i + 2)
        pl.when(i + 2 < BPW)(prefetch_idx)

        def drain_out():
            out_wait(k)
        pl.when(i >= 2)(drain_out)

        @plsc.parallel_loop(0, S, unroll=4)
        def addrow(r):
            for p in range(D // L):
                sl = pl.ds(p * L, L)
                ob[k][r, sl] = gb[k][r, sl] + pos_v[r, sl]

        out_start(k, b0 + i)

    def outer(g, c):
        iter_body(2 * g, 0)
        iter_body(2 * g + 1, 1)
        return c

    lax.fori_loop(0, BPW // 2, outer, 0)
    out_wait(0)
    out_wait(1)


@jax.jit
def kernel(input_ids, table, pos_embed):
    mesh = plsc.VectorSubcoreMesh(core_axis_name="c", subcore_axis_name="s")
    return pl.kernel(
        _emb_body,
        mesh=mesh,
        out_type=jax.ShapeDtypeStruct((B, S, D), jnp.float32),
        scratch_types=[
            pltpu.VMEM((S, D), jnp.float32),
            pltpu.VMEM((S,), jnp.int32),
            pltpu.VMEM((S,), jnp.int32),
            pltpu.VMEM((S, D), jnp.float32),
            pltpu.VMEM((S, D), jnp.float32),
            pltpu.VMEM((S, D), jnp.float32),
            pltpu.VMEM((S, D), jnp.float32),
            pltpu.SemaphoreType.DMA,
            pltpu.SemaphoreType.DMA,
            pltpu.SemaphoreType.DMA,
            pltpu.SemaphoreType.DMA,
            pltpu.SemaphoreType.DMA,
            pltpu.SemaphoreType.DMA,
        ],
    )(input_ids, pos_embed, table)
